# baseline (speedup 1.0000x reference)
import jax, jax.numpy as jnp
from jax.experimental import pallas as pl


def _conv(x, edge_index, W, b):
    n = x.shape[0]
    loop = jnp.arange(n, dtype=edge_index.dtype)
    src = jnp.concatenate([edge_index[0], loop])
    dst = jnp.concatenate([edge_index[1], loop])
    deg = jnp.zeros((n,), jnp.float32).at[dst].add(1.0)
    dinv = jnp.where(deg > 0, 1.0 / jnp.sqrt(deg), 0.0)
    norm = dinv[src] * dinv[dst]
    xw = x @ W
    msg = xw[src] * norm[:, None]
    out = jnp.zeros((n, W.shape[1]), jnp.float32).at[dst].add(msg)
    return out + b


def kernel(x, alpha, torque, edge_index, W_se, b_se, W_pe1, b_pe1, W_pe2, b_pe2, W_c1, b_c1, W_c2, b_c2, W_pc1, b_pc1, W_pc2, b_pc2, W_cc, b_cc, W_d1, b_d1, W_d2, b_d2, W_d3, b_d3):
    sf = jax.nn.sigmoid(x @ W_se + b_se)
    sf = jax.nn.relu(_conv(sf, edge_index, W_c1, b_c1))
    sf = jax.nn.relu(_conv(sf, edge_index, W_c2, b_c2))
    pin = jnp.concatenate([alpha, torque], axis=1)
    pf = jax.nn.sigmoid(pin @ W_pe1 + b_pe1)
    pf = jax.nn.sigmoid(pf @ W_pe2 + b_pe2)
    pf = jax.nn.relu(_conv(pf, edge_index, W_pc1, b_pc1))
    pf = jax.nn.relu(_conv(pf, edge_index, W_pc2, b_pc2))
    cf = jnp.concatenate([sf, pf], axis=1)
    cf = jax.nn.relu(_conv(cf, edge_index, W_cc, b_cc))
    d = jax.nn.sigmoid(cf @ W_d1 + b_d1)
    d = jax.nn.sigmoid(d @ W_d2 + b_d2)
    d = d @ W_d3 + b_d3
    return jax.nn.sigmoid(d)


# speedup vs baseline: 7.0012x; 7.0012x over previous
import functools

import jax
import jax.numpy as jnp
from jax import lax
from jax.experimental import pallas as pl
from jax.experimental.pallas import tpu as pltpu
from jax.experimental.pallas import tpu_sc as plsc

F32 = jnp.float32

N_NODES = 50000
H = 64
QUART = 16
N_ACC = 51200
E_EDGES = 800000
E_PAD = 802816
ROWS_PER_TILE = N_ACC // 16
PROP_CHUNKS = E_PAD // (16 * 128)
N_SEGS = 2
SEG_CHUNKS = PROP_CHUNKS // N_SEGS
DEG_CHUNKS = E_PAD // (2 * 16 * 128)
INIT_CHUNK = 400
INIT_STEPS = ROWS_PER_TILE // INIT_CHUNK
BM = 2048
GRID_I = N_ACC // BM

_mesh = plsc.VectorSubcoreMesh(
    core_axis_name="c", subcore_axis_name="s", num_cores=2, num_subcores=16)
_sc_params = pltpu.CompilerParams(use_tc_tiling_on_sc=False)


@functools.partial(
    pl.kernel,
    out_type=jax.ShapeDtypeStruct((2 * N_ACC,), F32),
    mesh=_mesh,
    scratch_types=[
        pltpu.VMEM((DEG_CHUNKS, 128), jnp.int32),
        pltpu.VMEM((128,), F32),
        pltpu.VMEM((ROWS_PER_TILE,), F32),
        pltpu.VMEM_SHARED((N_ACC,), F32),
    ],
    compiler_params=_sc_params,
)
def _deg_kernel(dst_hbm, ones_hbm, out_hbm, idx_v, ones_v, bounce_v, acc_sh):
    c = lax.axis_index("c")
    s = lax.axis_index("s")
    base = s * ROWS_PER_TILE
    pltpu.sync_copy(dst_hbm.at[c, s], idx_v)
    pltpu.sync_copy(ones_hbm, ones_v)

    @pl.loop(0, ROWS_PER_TILE // 16)
    def _(i):
        bounce_v[pl.ds(i * 16, 16)] = jnp.zeros((16,), F32)

    pltpu.sync_copy(bounce_v, acc_sh.at[pl.ds(base, ROWS_PER_TILE)])
    plsc.subcore_barrier()

    @pl.loop(0, DEG_CHUNKS)
    def _(j):
        pltpu.sync_copy(ones_v, acc_sh.at[idx_v.at[j]], add=True)

    plsc.subcore_barrier()
    pltpu.sync_copy(acc_sh.at[pl.ds(base, ROWS_PER_TILE)], bounce_v)
    pltpu.sync_copy(bounce_v,
                    out_hbm.at[pl.ds(c * N_ACC + base, ROWS_PER_TILE)])


@functools.partial(
    pl.kernel,
    out_type=jax.ShapeDtypeStruct((2 * N_ACC, QUART), F32),
    mesh=_mesh,
    scratch_types=[
        pltpu.VMEM((SEG_CHUNKS, 128), jnp.int32),
        pltpu.VMEM((SEG_CHUNKS, 128), jnp.int32),
        pltpu.VMEM((128, QUART), F32),
        pltpu.VMEM((INIT_CHUNK, QUART), F32),
        pltpu.VMEM_SHARED((N_ACC, QUART), F32),
        pltpu.SemaphoreType.DMA,
    ],
    compiler_params=_sc_params,
)
def _prop_kernel(z4_hbm, src_hbm, dst_hbm, out_hbm, src_v, dst_v, rows_v,
                 bounce_v, acc_sh, sem):
    c = lax.axis_index("c")
    s = lax.axis_index("s")
    base = s * ROWS_PER_TILE

    @pl.loop(0, INIT_CHUNK)
    def _(i):
        bounce_v[i] = jnp.zeros((QUART,), F32)

    @pl.loop(0, INIT_STEPS)
    def _(i):
        pltpu.sync_copy(bounce_v,
                        acc_sh.at[pl.ds(base + i * INIT_CHUNK, INIT_CHUNK)])

    plsc.subcore_barrier()

    for seg in range(N_SEGS):
        pltpu.sync_copy(src_hbm.at[c, s, seg], src_v)
        pltpu.sync_copy(dst_hbm.at[s, seg], dst_v)

        @pl.loop(0, SEG_CHUNKS)
        def _(j):
            pltpu.async_copy(z4_hbm.at[src_v.at[j]], rows_v, sem).wait()
            pltpu.sync_copy(rows_v, acc_sh.at[dst_v.at[j]], add=True)

    plsc.subcore_barrier()

    @pl.loop(0, INIT_STEPS)
    def _(i):
        pltpu.sync_copy(acc_sh.at[pl.ds(base + i * INIT_CHUNK, INIT_CHUNK)],
                        bounce_v)
        pltpu.sync_copy(
            bounce_v,
            out_hbm.at[pl.ds(c * N_ACC + base + i * INIT_CHUNK, INIT_CHUNK)])


def _fix_spec(shape):
    return pl.BlockSpec(shape, lambda *idx: (0,) * len(shape))


def _row_spec(w, ng=2):
    if ng == 1:
        return pl.BlockSpec((BM, w), lambda i: (i, 0))
    return pl.BlockSpec((BM, w), lambda i, qq: (i, 0))


def _y_specs(ng=2):
    if ng == 1:
        return [pl.BlockSpec((BM, QUART), lambda i, h=h: (h * GRID_I + i, 0))
                for h in range(2)]
    return [pl.BlockSpec((BM, QUART), lambda i, qq, h=h: (h * GRID_I + i, 0))
            for h in range(2)]


def _z_specs(ng=2):
    if ng == 1:
        return [pl.BlockSpec((BM, QUART), lambda i, q=q: (q * GRID_I + i, 0))
                for q in range(4)]
    return [pl.BlockSpec((BM, QUART), lambda i, qq, q=q: (q * GRID_I + i, 0))
            for q in range(4)]


def _z_out_spec():
    return pl.BlockSpec((BM, QUART), lambda i, qq: (qq * GRID_I + i, 0))


def _write_quarter(out_ref, z, qq):
    for q in range(4):
        @pl.when(qq == q)
        def _(q=q):
            out_ref[...] = z[:, q * QUART:(q + 1) * QUART]


def _t0_body(deg0, deg1, xin, Wse, bse, Wpe1, bpe1, Wpe2, bpe2, Wc1, Wpc1,
             zs_out, zp_out, dinv_out):
    qq = pl.program_id(1)
    dinv = lax.rsqrt(1.0 + deg0[...] + deg1[...])
    xb = xin[...]
    sf0 = jax.nn.sigmoid(jnp.dot(xb, Wse[...], preferred_element_type=F32)
                         + bse[...])
    ph = jax.nn.sigmoid(jnp.dot(xb, Wpe1[...], preferred_element_type=F32)
                        + bpe1[...])
    pf0 = jax.nn.sigmoid(jnp.dot(ph, Wpe2[...], preferred_element_type=F32)
                         + bpe2[...])
    dinv_out[...] = dinv
    _write_quarter(zs_out, dinv * jnp.dot(sf0, Wc1[...],
                                          preferred_element_type=F32), qq)
    _write_quarter(zp_out, dinv * jnp.dot(pf0, Wpc1[...],
                                          preferred_element_type=F32), qq)


def _t1_body(yA0, yA1, yB0, yB1, z0, z1, z2, z3, dinv, b_prev, Wn, z_out):
    qq = pl.program_id(1)
    dv = dinv[...]
    y = jnp.concatenate([yA0[...], yA1[...], yB0[...], yB1[...]], axis=1)
    z = jnp.concatenate([z0[...], z1[...], z2[...], z3[...]], axis=1)
    h = jnp.maximum(dv * (y + z) + b_prev[...], 0.0)
    _write_quarter(z_out, dv * jnp.dot(h, Wn[...],
                                       preferred_element_type=F32), qq)


def _t2_body(sA0, sA1, sB0, sB1, zs0, zs1, zs2, zs3,
             pA0, pA1, pB0, pB1, zp0, zp1, zp2, zp3,
             dinv, bc2, bpc2, Wcc, z_out):
    qq = pl.program_id(1)
    dv = dinv[...]
    ys = jnp.concatenate([sA0[...], sA1[...], sB0[...], sB1[...]], axis=1)
    zs = jnp.concatenate([zs0[...], zs1[...], zs2[...], zs3[...]], axis=1)
    yp = jnp.concatenate([pA0[...], pA1[...], pB0[...], pB1[...]], axis=1)
    zp = jnp.concatenate([zp0[...], zp1[...], zp2[...], zp3[...]], axis=1)
    sf2 = jnp.maximum(dv * (ys + zs) + bc2[...], 0.0)
    pf2 = jnp.maximum(dv * (yp + zp) + bpc2[...], 0.0)
    W = Wcc[...]
    z = dv * (jnp.dot(sf2, W[:H], preferred_element_type=F32)
              + jnp.dot(pf2, W[H:], preferred_element_type=F32))
    _write_quarter(z_out, z, qq)


def _t3_body(yA0, yA1, yB0, yB1, z0, z1, z2, z3, dinv, bcc,
             Wd1, bd1, Wd2, bd2, Wd3, bd3, out):
    dv = dinv[...]
    y = jnp.concatenate([yA0[...], yA1[...], yB0[...], yB1[...]], axis=1)
    z = jnp.concatenate([z0[...], z1[...], z2[...], z3[...]], axis=1)
    cf = jnp.maximum(dv * (y + z) + bcc[...], 0.0)
    d = jax.nn.sigmoid(jnp.dot(cf, Wd1[...], preferred_element_type=F32)
                       + bd1[...])
    d = jax.nn.sigmoid(jnp.dot(d, Wd2[...], preferred_element_type=F32)
                       + bd2[...])
    d = jnp.dot(d, Wd3[...], preferred_element_type=F32) + bd3[...]
    out[...] = jax.nn.sigmoid(d)


def kernel(x, alpha, torque, edge_index, W_se, b_se, W_pe1, b_pe1, W_pe2,
           b_pe2, W_c1, b_c1, W_c2, b_c2, W_pc1, b_pc1, W_pc2, b_pc2, W_cc,
           b_cc, W_d1, b_d1, W_d2, b_d2, W_d3, b_d3):
    pad = E_PAD - E_EDGES
    src_p = jnp.concatenate([edge_index[0], jnp.zeros((pad,), jnp.int32)])
    dst_p = jnp.concatenate([edge_index[1],
                             jnp.full((pad,), N_NODES, jnp.int32)])
    src3A = jnp.stack([src_p, src_p + N_ACC]).reshape(
        2, 16, N_SEGS, SEG_CHUNKS, 128)
    src3B = jnp.stack([src_p + 2 * N_ACC, src_p + 3 * N_ACC]).reshape(
        2, 16, N_SEGS, SEG_CHUNKS, 128)
    dst3 = dst_p.reshape(16, N_SEGS, SEG_CHUNKS, 128)
    dst4 = dst_p.reshape(2, 16, DEG_CHUNKS, 128)
    ones_h = jnp.ones((128,), F32)

    xin = (jnp.zeros((N_ACC, 8), F32)
           .at[:N_NODES, 0:3].set(x)
           .at[:N_NODES, 3:4].set(alpha)
           .at[:N_NODES, 4:5].set(torque))
    Wse8 = jnp.zeros((8, H), F32).at[0:3].set(W_se)
    Wpe8 = jnp.zeros((8, H), F32).at[3:5].set(W_pe1)

    deg2 = _deg_kernel(dst4, ones_h).reshape(2 * N_ACC, 1)

    z4_sds = jax.ShapeDtypeStruct((4 * N_ACC, QUART), F32)

    zs4, zp4, dinv = pl.pallas_call(
        _t0_body,
        grid=(GRID_I, 4),
        in_specs=[
            _row_spec(1),
            pl.BlockSpec((BM, 1), lambda i, qq: (GRID_I + i, 0)),
            _row_spec(8),
            _fix_spec((8, H)), _fix_spec((1, H)),
            _fix_spec((8, H)), _fix_spec((1, H)),
            _fix_spec((H, H)), _fix_spec((1, H)),
            _fix_spec((H, H)), _fix_spec((H, H)),
        ],
        out_specs=[_z_out_spec(), _z_out_spec(),
                   pl.BlockSpec((BM, 1), lambda i, qq: (i, 0))],
        out_shape=[z4_sds, z4_sds, jax.ShapeDtypeStruct((N_ACC, 1), F32)],
    )(deg2, deg2, xin, Wse8, b_se.reshape(1, H), Wpe8, b_pe1.reshape(1, H),
      W_pe2, b_pe2.reshape(1, H), W_c1, W_pc1)

    def prop(z4):
        yA = _prop_kernel(z4, src3A, dst3)
        yB = _prop_kernel(z4, src3B, dst3)
        return yA, yB

    def t1(yA, yB, z4, b_prev, Wn):
        return pl.pallas_call(
            _t1_body,
            grid=(GRID_I, 4),
            in_specs=([_y_specs()[0], _y_specs()[1], _y_specs()[0],
                       _y_specs()[1]] + _z_specs()
                      + [_row_spec(1), _fix_spec((1, H)), _fix_spec((H, H))]),
            out_specs=_z_out_spec(),
            out_shape=z4_sds,
        )(yA, yA, yB, yB, z4, z4, z4, z4, dinv, b_prev.reshape(1, H), Wn)

    ysA1, ysB1 = prop(zs4)
    ypA1, ypB1 = prop(zp4)
    zs4_2 = t1(ysA1, ysB1, zs4, b_c1, W_c2)
    zp4_2 = t1(ypA1, ypB1, zp4, b_pc1, W_pc2)
    ysA2, ysB2 = prop(zs4_2)
    ypA2, ypB2 = prop(zp4_2)

    zcc4 = pl.pallas_call(
        _t2_body,
        grid=(GRID_I, 4),
        in_specs=([_y_specs()[0], _y_specs()[1], _y_specs()[0],
                   _y_specs()[1]] + _z_specs()
                  + [_y_specs()[0], _y_specs()[1], _y_specs()[0],
                     _y_specs()[1]] + _z_specs()
                  + [_row_spec(1), _fix_spec((1, H)), _fix_spec((1, H)),
                     _fix_spec((2 * H, H))]),
        out_specs=_z_out_spec(),
        out_shape=z4_sds,
    )(ysA2, ysA2, ysB2, ysB2, zs4_2, zs4_2, zs4_2, zs4_2,
      ypA2, ypA2, ypB2, ypB2, zp4_2, zp4_2, zp4_2, zp4_2,
      dinv, b_c2.reshape(1, H), b_pc2.reshape(1, H), W_cc)

    yccA, yccB = prop(zcc4)

    y1 = _y_specs(ng=1)
    damping = pl.pallas_call(
        _t3_body,
        grid=(GRID_I,),
        in_specs=([y1[0], y1[1], y1[0], y1[1]] + _z_specs(ng=1)
                  + [_row_spec(1, ng=1), _fix_spec((1, H)),
                     _fix_spec((H, H)), _fix_spec((1, H)),
                     _fix_spec((H, H // 2)), _fix_spec((1, H // 2)),
                     _fix_spec((H // 2, 1)), _fix_spec((1, 1))]),
        out_specs=pl.BlockSpec((BM, 1), lambda i: (i, 0)),
        out_shape=jax.ShapeDtypeStruct((N_ACC, 1), F32),
    )(yccA, yccA, yccB, yccB, zcc4, zcc4, zcc4, zcc4, dinv,
      b_cc.reshape(1, H), W_d1, b_d1.reshape(1, H),
      W_d2, b_d2.reshape(1, H // 2), W_d3, b_d3.reshape(1, 1))

    return damping[:N_NODES]


# speedup vs baseline: 10.8320x; 1.5472x over previous
import functools

import jax
import jax.numpy as jnp
from jax import lax
from jax.experimental import pallas as pl
from jax.experimental.pallas import tpu as pltpu
from jax.experimental.pallas import tpu_sc as plsc

F32 = jnp.float32

N_NODES = 50000
H = 64
QUART = 16
N_ACC = 51200
E_EDGES = 800000
E_PAD = 802816
ROWS_PER_TILE = N_ACC // 16
PROP_CHUNKS = E_PAD // (16 * 128)
N_SEGS = 2
PIPE = 4
SEG_CHUNKS = PROP_CHUNKS // N_SEGS
DEG_CHUNKS = E_PAD // (2 * 16 * 128)
INIT_CHUNK = 400
INIT_STEPS = ROWS_PER_TILE // INIT_CHUNK
BM = 2048
GRID_I = N_ACC // BM

_mesh = plsc.VectorSubcoreMesh(
    core_axis_name="c", subcore_axis_name="s", num_cores=2, num_subcores=16)
_sc_params = pltpu.CompilerParams(use_tc_tiling_on_sc=False)


@functools.partial(
    pl.kernel,
    out_type=jax.ShapeDtypeStruct((2 * N_ACC,), F32),
    mesh=_mesh,
    scratch_types=[
        pltpu.VMEM((DEG_CHUNKS, 128), jnp.int32),
        pltpu.VMEM((128,), F32),
        pltpu.VMEM((ROWS_PER_TILE,), F32),
        pltpu.VMEM_SHARED((N_ACC,), F32),
        pltpu.SemaphoreType.DMA((PIPE,)),
    ],
    compiler_params=_sc_params,
)
def _deg_kernel(dst_hbm, ones_hbm, out_hbm, idx_v, ones_v, bounce_v, acc_sh,
                sem):
    c = lax.axis_index("c")
    s = lax.axis_index("s")
    base = s * ROWS_PER_TILE
    pltpu.sync_copy(dst_hbm.at[c, s], idx_v)
    pltpu.sync_copy(ones_hbm, ones_v)

    @pl.loop(0, ROWS_PER_TILE // 16)
    def _(i):
        bounce_v[pl.ds(i * 16, 16)] = jnp.zeros((16,), F32)

    pltpu.sync_copy(bounce_v, acc_sh.at[pl.ds(base, ROWS_PER_TILE)])
    plsc.subcore_barrier()

    @pl.loop(0, DEG_CHUNKS // PIPE)
    def _(g):
        j0 = g * PIPE
        descs = [
            pltpu.async_copy(ones_v, acc_sh.at[idx_v.at[j0 + k]], sem.at[k],
                             add=True)
            for k in range(PIPE)
        ]
        for k in range(PIPE):
            descs[k].wait()

    plsc.subcore_barrier()
    pltpu.sync_copy(acc_sh.at[pl.ds(base, ROWS_PER_TILE)],
                    out_hbm.at[pl.ds(c * N_ACC + base, ROWS_PER_TILE)])


@functools.partial(
    pl.kernel,
    out_type=jax.ShapeDtypeStruct((2 * N_ACC, QUART), F32),
    mesh=_mesh,
    scratch_types=[
        pltpu.VMEM((SEG_CHUNKS, 128), jnp.int32),
        pltpu.VMEM((SEG_CHUNKS, 128), jnp.int32),
        pltpu.VMEM((PIPE, 128, QUART), F32),
        pltpu.VMEM((INIT_CHUNK, QUART), F32),
        pltpu.VMEM_SHARED((N_ACC, QUART), F32),
        pltpu.SemaphoreType.DMA((PIPE,)),
    ],
    compiler_params=_sc_params,
)
def _prop_kernel(z4_hbm, src_hbm, dst_hbm, out_hbm, src_v, dst_v, rows_v,
                 bounce_v, acc_sh, sem):
    c = lax.axis_index("c")
    s = lax.axis_index("s")
    base = s * ROWS_PER_TILE

    @pl.loop(0, INIT_CHUNK)
    def _(i):
        bounce_v[i] = jnp.zeros((QUART,), F32)

    @pl.loop(0, INIT_STEPS)
    def _(i):
        pltpu.sync_copy(bounce_v,
                        acc_sh.at[pl.ds(base + i * INIT_CHUNK, INIT_CHUNK)])

    plsc.subcore_barrier()

    for seg in range(N_SEGS):
        pltpu.sync_copy(src_hbm.at[c, s, seg], src_v)
        pltpu.sync_copy(dst_hbm.at[s, seg], dst_v)

        @pl.loop(0, SEG_CHUNKS // PIPE)
        def _(g):
            j0 = g * PIPE
            descs = [
                pltpu.async_copy(z4_hbm.at[src_v.at[j0 + k]], rows_v.at[k],
                                 sem.at[k])
                for k in range(PIPE)
            ]
            for k in range(PIPE):
                descs[k].wait()
                pltpu.sync_copy(rows_v.at[k], acc_sh.at[dst_v.at[j0 + k]],
                                add=True)

    plsc.subcore_barrier()

    pltpu.sync_copy(acc_sh.at[pl.ds(base, ROWS_PER_TILE)],
                    out_hbm.at[pl.ds(c * N_ACC + base, ROWS_PER_TILE)])


def _fix_spec(shape):
    return pl.BlockSpec(shape, lambda *idx: (0,) * len(shape))


def _row_spec(w, ng=2):
    if ng == 1:
        return pl.BlockSpec((BM, w), lambda i: (i, 0))
    return pl.BlockSpec((BM, w), lambda i, qq: (i, 0))


def _y_specs(ng=2):
    if ng == 1:
        return [pl.BlockSpec((BM, QUART), lambda i, h=h: (h * GRID_I + i, 0))
                for h in range(2)]
    return [pl.BlockSpec((BM, QUART), lambda i, qq, h=h: (h * GRID_I + i, 0))
            for h in range(2)]


def _z_specs(ng=2):
    if ng == 1:
        return [pl.BlockSpec((BM, QUART), lambda i, q=q: (q * GRID_I + i, 0))
                for q in range(4)]
    return [pl.BlockSpec((BM, QUART), lambda i, qq, q=q: (q * GRID_I + i, 0))
            for q in range(4)]


def _z_out_spec():
    return pl.BlockSpec((BM, QUART), lambda i, qq: (qq * GRID_I + i, 0))


def _write_quarter(out_ref, z, qq):
    for q in range(4):
        @pl.when(qq == q)
        def _(q=q):
            out_ref[...] = z[:, q * QUART:(q + 1) * QUART]


def _t0_body(deg0, deg1, xin, Wse, bse, Wpe1, bpe1, Wpe2, bpe2, Wc1, Wpc1,
             zs_out, zp_out, dinv_out):
    qq = pl.program_id(1)
    dinv = lax.rsqrt(1.0 + deg0[...] + deg1[...])
    xb = xin[...]
    sf0 = jax.nn.sigmoid(jnp.dot(xb, Wse[...], preferred_element_type=F32)
                         + bse[...])
    ph = jax.nn.sigmoid(jnp.dot(xb, Wpe1[...], preferred_element_type=F32)
                        + bpe1[...])
    pf0 = jax.nn.sigmoid(jnp.dot(ph, Wpe2[...], preferred_element_type=F32)
                         + bpe2[...])
    dinv_out[...] = dinv
    _write_quarter(zs_out, dinv * jnp.dot(sf0, Wc1[...],
                                          preferred_element_type=F32), qq)
    _write_quarter(zp_out, dinv * jnp.dot(pf0, Wpc1[...],
                                          preferred_element_type=F32), qq)


def _t1_body(yA0, yA1, yB0, yB1, z0, z1, z2, z3, dinv, b_prev, Wn, z_out):
    qq = pl.program_id(1)
    dv = dinv[...]
    y = jnp.concatenate([yA0[...], yA1[...], yB0[...], yB1[...]], axis=1)
    z = jnp.concatenate([z0[...], z1[...], z2[...], z3[...]], axis=1)
    h = jnp.maximum(dv * (y + z) + b_prev[...], 0.0)
    _write_quarter(z_out, dv * jnp.dot(h, Wn[...],
                                       preferred_element_type=F32), qq)


def _t2_body(sA0, sA1, sB0, sB1, zs0, zs1, zs2, zs3,
             pA0, pA1, pB0, pB1, zp0, zp1, zp2, zp3,
             dinv, bc2, bpc2, Wcc, z_out):
    qq = pl.program_id(1)
    dv = dinv[...]
    ys = jnp.concatenate([sA0[...], sA1[...], sB0[...], sB1[...]], axis=1)
    zs = jnp.concatenate([zs0[...], zs1[...], zs2[...], zs3[...]], axis=1)
    yp = jnp.concatenate([pA0[...], pA1[...], pB0[...], pB1[...]], axis=1)
    zp = jnp.concatenate([zp0[...], zp1[...], zp2[...], zp3[...]], axis=1)
    sf2 = jnp.maximum(dv * (ys + zs) + bc2[...], 0.0)
    pf2 = jnp.maximum(dv * (yp + zp) + bpc2[...], 0.0)
    W = Wcc[...]
    z = dv * (jnp.dot(sf2, W[:H], preferred_element_type=F32)
              + jnp.dot(pf2, W[H:], preferred_element_type=F32))
    _write_quarter(z_out, z, qq)


def _t3_body(yA0, yA1, yB0, yB1, z0, z1, z2, z3, dinv, bcc,
             Wd1, bd1, Wd2, bd2, Wd3, bd3, out):
    dv = dinv[...]
    y = jnp.concatenate([yA0[...], yA1[...], yB0[...], yB1[...]], axis=1)
    z = jnp.concatenate([z0[...], z1[...], z2[...], z3[...]], axis=1)
    cf = jnp.maximum(dv * (y + z) + bcc[...], 0.0)
    d = jax.nn.sigmoid(jnp.dot(cf, Wd1[...], preferred_element_type=F32)
                       + bd1[...])
    d = jax.nn.sigmoid(jnp.dot(d, Wd2[...], preferred_element_type=F32)
                       + bd2[...])
    d = jnp.dot(d, Wd3[...], preferred_element_type=F32) + bd3[...]
    out[...] = jax.nn.sigmoid(d)


def kernel(x, alpha, torque, edge_index, W_se, b_se, W_pe1, b_pe1, W_pe2,
           b_pe2, W_c1, b_c1, W_c2, b_c2, W_pc1, b_pc1, W_pc2, b_pc2, W_cc,
           b_cc, W_d1, b_d1, W_d2, b_d2, W_d3, b_d3):
    pad = E_PAD - E_EDGES
    src_p = jnp.concatenate([edge_index[0], jnp.zeros((pad,), jnp.int32)])
    dst_p = jnp.concatenate([edge_index[1],
                             jnp.full((pad,), N_NODES, jnp.int32)])
    src3A = jnp.stack([src_p, src_p + N_ACC]).reshape(
        2, 16, N_SEGS, SEG_CHUNKS, 128)
    src3B = jnp.stack([src_p + 2 * N_ACC, src_p + 3 * N_ACC]).reshape(
        2, 16, N_SEGS, SEG_CHUNKS, 128)
    dst3 = dst_p.reshape(16, N_SEGS, SEG_CHUNKS, 128)
    dst4 = dst_p.reshape(2, 16, DEG_CHUNKS, 128)
    ones_h = jnp.ones((128,), F32)

    xin = (jnp.zeros((N_ACC, 8), F32)
           .at[:N_NODES, 0:3].set(x)
           .at[:N_NODES, 3:4].set(alpha)
           .at[:N_NODES, 4:5].set(torque))
    Wse8 = jnp.zeros((8, H), F32).at[0:3].set(W_se)
    Wpe8 = jnp.zeros((8, H), F32).at[3:5].set(W_pe1)

    deg2 = _deg_kernel(dst4, ones_h).reshape(2 * N_ACC, 1)

    z4_sds = jax.ShapeDtypeStruct((4 * N_ACC, QUART), F32)

    zs4, zp4, dinv = pl.pallas_call(
        _t0_body,
        grid=(GRID_I, 4),
        in_specs=[
            _row_spec(1),
            pl.BlockSpec((BM, 1), lambda i, qq: (GRID_I + i, 0)),
            _row_spec(8),
            _fix_spec((8, H)), _fix_spec((1, H)),
            _fix_spec((8, H)), _fix_spec((1, H)),
            _fix_spec((H, H)), _fix_spec((1, H)),
            _fix_spec((H, H)), _fix_spec((H, H)),
        ],
        out_specs=[_z_out_spec(), _z_out_spec(),
                   pl.BlockSpec((BM, 1), lambda i, qq: (i, 0))],
        out_shape=[z4_sds, z4_sds, jax.ShapeDtypeStruct((N_ACC, 1), F32)],
    )(deg2, deg2, xin, Wse8, b_se.reshape(1, H), Wpe8, b_pe1.reshape(1, H),
      W_pe2, b_pe2.reshape(1, H), W_c1, W_pc1)

    def prop(z4):
        yA = _prop_kernel(z4, src3A, dst3)
        yB = _prop_kernel(z4, src3B, dst3)
        return yA, yB

    def t1(yA, yB, z4, b_prev, Wn):
        return pl.pallas_call(
            _t1_body,
            grid=(GRID_I, 4),
            in_specs=([_y_specs()[0], _y_specs()[1], _y_specs()[0],
                       _y_specs()[1]] + _z_specs()
                      + [_row_spec(1), _fix_spec((1, H)), _fix_spec((H, H))]),
            out_specs=_z_out_spec(),
            out_shape=z4_sds,
        )(yA, yA, yB, yB, z4, z4, z4, z4, dinv, b_prev.reshape(1, H), Wn)

    ysA1, ysB1 = prop(zs4)
    ypA1, ypB1 = prop(zp4)
    zs4_2 = t1(ysA1, ysB1, zs4, b_c1, W_c2)
    zp4_2 = t1(ypA1, ypB1, zp4, b_pc1, W_pc2)
    ysA2, ysB2 = prop(zs4_2)
    ypA2, ypB2 = prop(zp4_2)

    zcc4 = pl.pallas_call(
        _t2_body,
        grid=(GRID_I, 4),
        in_specs=([_y_specs()[0], _y_specs()[1], _y_specs()[0],
                   _y_specs()[1]] + _z_specs()
                  + [_y_specs()[0], _y_specs()[1], _y_specs()[0],
                     _y_specs()[1]] + _z_specs()
                  + [_row_spec(1), _fix_spec((1, H)), _fix_spec((1, H)),
                     _fix_spec((2 * H, H))]),
        out_specs=_z_out_spec(),
        out_shape=z4_sds,
    )(ysA2, ysA2, ysB2, ysB2, zs4_2, zs4_2, zs4_2, zs4_2,
      ypA2, ypA2, ypB2, ypB2, zp4_2, zp4_2, zp4_2, zp4_2,
      dinv, b_c2.reshape(1, H), b_pc2.reshape(1, H), W_cc)

    yccA, yccB = prop(zcc4)

    y1 = _y_specs(ng=1)
    damping = pl.pallas_call(
        _t3_body,
        grid=(GRID_I,),
        in_specs=([y1[0], y1[1], y1[0], y1[1]] + _z_specs(ng=1)
                  + [_row_spec(1, ng=1), _fix_spec((1, H)),
                     _fix_spec((H, H)), _fix_spec((1, H)),
                     _fix_spec((H, H // 2)), _fix_spec((1, H // 2)),
                     _fix_spec((H // 2, 1)), _fix_spec((1, 1))]),
        out_specs=pl.BlockSpec((BM, 1), lambda i: (i, 0)),
        out_shape=jax.ShapeDtypeStruct((N_ACC, 1), F32),
    )(yccA, yccA, yccB, yccB, zcc4, zcc4, zcc4, zcc4, dinv,
      b_cc.reshape(1, H), W_d1, b_d1.reshape(1, H),
      W_d2, b_d2.reshape(1, H // 2), W_d3, b_d3.reshape(1, 1))

    return damping[:N_NODES]
